# baseline (speedup 1.0000x reference)
import functools

import jax
import jax.numpy as jnp
from jax import lax
from jax.experimental import pallas as pl
from jax.experimental.pallas import tpu as pltpu
from jax.experimental.pallas import tpu_sc as plsc

D_MODEL = 32
_ROWS, _COLS = 16384, 26
_B = _ROWS * _COLS
_L = 128
_U = 256
_UPR = _ROWS // _U
_NW = 32
_NU = _COLS * _UPR // _NW

_IT = _ROWS // _L
_IT_PW = _IT // _NW


def _make_flatten():
    mesh = plsc.VectorSubcoreMesh(core_axis_name="c", subcore_axis_name="s")

    @functools.partial(
        pl.kernel,
        out_type=jax.ShapeDtypeStruct((_B,), jnp.float32),
        mesh=mesh,
        scratch_types=[
            pltpu.VMEM((8, _L), jnp.float32),
        ],
        compiler_params=pltpu.CompilerParams(use_tc_tiling_on_sc=True),
    )
    def flatten(xt_hbm, out_hbm, blk_v):
        wid = lax.axis_index("s") * 2 + lax.axis_index("c")
        for jg, h in ((0, 8), (1, 8), (2, 8), (3, 2)):
            for t in range(_IT_PW):
                it = wid * _IT_PW + t
                pltpu.sync_copy(
                    xt_hbm.at[pl.ds(jg * 8, h), pl.ds(it * _L, _L)],
                    blk_v.at[pl.ds(0, h)])
                for r in range(h):
                    pltpu.sync_copy(
                        blk_v.at[r],
                        out_hbm.at[pl.ds((jg * 8 + r) * _ROWS + it * _L,
                                         _L)])

    return flatten


def _make_gather():
    mesh = plsc.VectorSubcoreMesh(core_axis_name="c", subcore_axis_name="s")

    @functools.partial(
        pl.kernel,
        out_type=jax.ShapeDtypeStruct((_COLS, 4, _IT, 8, _L), jnp.float32),
        mesh=mesh,
        scratch_types=[
            pltpu.VMEM((2, _U), jnp.float32),
            pltpu.VMEM((2, _U), jnp.int32),
            pltpu.VMEM((2, _U), jnp.int32),
            pltpu.VMEM((2, _U, _L), jnp.float32),
            pltpu.VMEM((2, 4, 2, 8, _L), jnp.float32),
            pltpu.SemaphoreType.DMA,
            pltpu.SemaphoreType.DMA,
            pltpu.SemaphoreType.DMA,
            pltpu.SemaphoreType.DMA,
        ],
        compiler_params=pltpu.CompilerParams(use_tc_tiling_on_sc=False,
                                             needs_layout_passes=False),
    )
    def gather(table_hbm, idx_hbm, out_hbm, idxf_v, br_v, sub_v, big_v,
               out_v, gsem0, gsem1, osem0, osem1):
        wid = lax.axis_index("s") * 2 + lax.axis_index("c")
        ubase = wid * _NU
        gsems = (gsem0, gsem1)
        osems = (osem0, osem1)

        @pl.loop(0, _NU, step=2)
        def _pair(uo):
            for b in range(2):
                u = ubase + uo + b
                j = u // _UPR
                it0 = (u % _UPR) * (_U // _L)

                @pl.when(uo != 0)
                def _reclaim():
                    for ch in range(4):
                        pltpu.make_async_copy(
                            out_v.at[b, ch],
                            out_hbm.at[j, ch, pl.ds(it0, _U // _L)],
                            osems[b]).wait()

                pltpu.sync_copy(
                    idx_hbm.at[pl.ds(j * _ROWS + it0 * _L, _U)],
                    idxf_v.at[b])
                for k in range(_U // 16):
                    v = plsc.bitcast(idxf_v[b, pl.ds(k * 16, 16)], jnp.int32)
                    br_v[b, pl.ds(k * 16, 16)] = lax.shift_right_logical(v, 2)
                    sub_v[b, pl.ds(k * 16, 16)] = lax.shift_left(v & 3, 5)
                for k in range(_U // _L):
                    pltpu.async_copy(
                        table_hbm.at[br_v.at[b, pl.ds(k * _L, _L)]],
                        big_v.at[b, pl.ds(k * _L, _L)],
                        gsems[b])
            for b in range(2):
                u = ubase + uo + b
                j = u // _UPR
                it0 = (u % _UPR) * (_U // _L)
                for k in range(_U // _L):
                    pltpu.make_async_copy(
                        table_hbm.at[br_v.at[b, pl.ds(k * _L, _L)]],
                        big_v.at[b, pl.ds(k * _L, _L)],
                        gsems[b]).wait()
                for nb in range(_U // 16):
                    rowv = lax.iota(jnp.int32, 16) + (nb * 16)
                    subv = sub_v[b, pl.ds(nb * 16, 16)]
                    for c in range(D_MODEL):
                        vals = plsc.load_gather(big_v.at[b],
                                                [rowv, subv + c])
                        out_v[b, c // 8, nb // 8, c % 8,
                              pl.ds((nb % 8) * 16, 16)] = vals
                for ch in range(4):
                    pltpu.async_copy(out_v.at[b, ch],
                                     out_hbm.at[j, ch, pl.ds(it0, _U // _L)],
                                     osems[b])

        for b in range(2):
            u = ubase + _NU - 2 + b
            j = u // _UPR
            it0 = (u % _UPR) * (_U // _L)
            for ch in range(4):
                pltpu.make_async_copy(
                    out_v.at[b, ch],
                    out_hbm.at[j, ch, pl.ds(it0, _U // _L)],
                    osems[b]).wait()

    return gather


_flatten = _make_flatten()
_gather = _make_gather()


@jax.jit
def kernel(x, embed_weight):
    xt = jnp.transpose(lax.bitcast_convert_type(x.astype(jnp.int32),
                                                jnp.float32))
    idx = _flatten(xt)
    table = embed_weight.reshape(250000, 128)
    out5 = _gather(table, idx)
    return out5.transpose(2, 4, 0, 1, 3).reshape(_ROWS, _COLS, D_MODEL)


# speedup vs baseline: 1.2720x; 1.2720x over previous
import functools

import jax
import jax.numpy as jnp
from jax import lax
from jax.experimental import pallas as pl
from jax.experimental.pallas import tpu as pltpu
from jax.experimental.pallas import tpu_sc as plsc

D_MODEL = 32
_ROWS, _COLS = 16384, 26
_B = _ROWS * _COLS
_L = 128
_NCHUNK = 4
_U = _NCHUNK * _L
_UPR = _ROWS // _U
_NW = 32
_NU = _COLS * _UPR // _NW


def _make_gather():
    mesh = plsc.VectorSubcoreMesh(core_axis_name="c", subcore_axis_name="s")

    @functools.partial(
        pl.kernel,
        out_type=jax.ShapeDtypeStruct((_COLS, _ROWS, D_MODEL), jnp.float32),
        mesh=mesh,
        scratch_types=[
            pltpu.VMEM((2, _U), jnp.float32),
            pltpu.VMEM((2, _U), jnp.int32),
            pltpu.VMEM((2, _U, D_MODEL), jnp.float32),
            pltpu.SemaphoreType.DMA,
            pltpu.SemaphoreType.DMA,
            pltpu.SemaphoreType.DMA,
            pltpu.SemaphoreType.DMA,
        ],
        compiler_params=pltpu.CompilerParams(use_tc_tiling_on_sc=False,
                                             needs_layout_passes=False),
    )
    def gather(table_hbm, idx_hbm, out_hbm, idxf_v, idx_v, rows_v,
               gsem0, gsem1, osem0, osem1):
        wid = lax.axis_index("s") * 2 + lax.axis_index("c")
        ubase = wid * _NU
        gsems = (gsem0, gsem1)
        osems = (osem0, osem1)

        @pl.loop(0, _NU, step=2)
        def _pair(uo):
            for b in range(2):
                u = ubase + uo + b
                j = u // _UPR
                i0 = (u % _UPR) * _U

                @pl.when(uo != 0)
                def _reclaim():
                    pltpu.make_async_copy(
                        rows_v.at[b],
                        out_hbm.at[j, pl.ds(i0, _U)],
                        osems[b]).wait()

                pltpu.sync_copy(idx_hbm.at[j, pl.ds(i0, _U)], idxf_v.at[b])
                for k in range(_U // 16):
                    idx_v[b, pl.ds(k * 16, 16)] = plsc.bitcast(
                        idxf_v[b, pl.ds(k * 16, 16)], jnp.int32)
                for k in range(_NCHUNK):
                    pltpu.async_copy(
                        table_hbm.at[idx_v.at[b, pl.ds(k * _L, _L)]],
                        rows_v.at[b, pl.ds(k * _L, _L)],
                        gsems[b])
            for b in range(2):
                u = ubase + uo + b
                j = u // _UPR
                i0 = (u % _UPR) * _U
                for k in range(_NCHUNK):
                    pltpu.make_async_copy(
                        table_hbm.at[idx_v.at[b, pl.ds(k * _L, _L)]],
                        rows_v.at[b, pl.ds(k * _L, _L)],
                        gsems[b]).wait()
                pltpu.async_copy(rows_v.at[b],
                                 out_hbm.at[j, pl.ds(i0, _U)],
                                 osems[b])

        for b in range(2):
            u = ubase + _NU - 2 + b
            j = u // _UPR
            i0 = (u % _UPR) * _U
            pltpu.make_async_copy(
                rows_v.at[b],
                out_hbm.at[j, pl.ds(i0, _U)],
                osems[b]).wait()

    return gather


_gather = _make_gather()


@jax.jit
def kernel(x, embed_weight):
    idx = jnp.transpose(lax.bitcast_convert_type(x.astype(jnp.int32),
                                                 jnp.float32))
    out = _gather(embed_weight, idx)
    return jnp.transpose(out, (1, 0, 2))
